# baseline (speedup 1.0000x reference)
import functools

import jax
import jax.numpy as jnp
from jax import lax
from jax.experimental import pallas as pl
from jax.experimental.pallas import tpu as pltpu
from jax.experimental.pallas import tpu_sc as plsc

EMBED_DIM = 64
S = 50
NUM_WORKERS = 32
BATCH = 4096
BB = BATCH // NUM_WORKERS
B_PER_W = BB * S
G = 5
NBATCH = S // G

_mesh = plsc.VectorSubcoreMesh(core_axis_name="c", subcore_axis_name="s")


@functools.partial(
    pl.kernel,
    mesh=_mesh,
    out_type=jax.ShapeDtypeStruct((S, 8, NUM_WORKERS, 8, 128), jnp.float32),
    scratch_types=(
        [pltpu.VMEM((B_PER_W,), jnp.int32)]
        + [pltpu.VMEM((G * BB,), jnp.int32) for _ in range(2)]
        + [pltpu.VMEM((G * BB, EMBED_DIM), jnp.float32) for _ in range(2)]
        + [pltpu.VMEM((8, 8, 128), jnp.float32) for _ in range(2)]
        + [pltpu.SemaphoreType.DMA for _ in range(4)]
    ),
    compiler_params=pltpu.CompilerParams(
        use_tc_tiling_on_sc=False, needs_layout_passes=False
    ),
)
def _gather_kernel(idx_hbm, table_hbm, out_hbm,
                   idx_v, stage_a, stage_b, row_a, row_b, outv_a, outv_b,
                   gsem_a, gsem_b, osem_a, osem_b):
    w = lax.axis_index("s") * 2 + lax.axis_index("c")

    pltpu.sync_copy(idx_hbm.at[pl.ds(w * B_PER_W, B_PER_W)], idx_v)

    iota = lax.iota(jnp.int32, 16)
    jm50 = [(jnp.full((16,), g * 16, jnp.int32) + iota) * S for g in range(8)]

    outvs = (outv_a, outv_b)
    osems = (osem_a, osem_b)

    def gather_batch(b0, stage, rowbuf, gsem):
        with jax.named_scope("stage"):
            for u in range(G):
                s_splat = jnp.full((16,), b0 + u, jnp.int32)
                for g in range(8):
                    v = plsc.load_gather(idx_v, [jm50[g] + s_splat])
                    stage[pl.ds((u * 8 + g) * 16, 16)] = v
            return pltpu.async_copy(table_hbm.at[stage], rowbuf, gsem)

    def drain_gather(stage, rowbuf, gsem):
        with jax.named_scope("dgr"):
            pltpu.make_async_copy(table_hbm.at[stage], rowbuf, gsem).wait()

    def transpose_unit(u, rowbuf, outv):
        @plsc.parallel_loop(0, EMBED_DIM, step=1, unroll=8)
        def body(d):
            cold = jnp.full((16,), d, jnp.int32)
            for g in range(8):
                row16 = jnp.full((16,), u * BB + g * 16, jnp.int32) + iota
                v = plsc.load_gather(rowbuf, [row16, cold])
                outv[d // 8, d % 8, pl.ds(g * 16, 16)] = v

    def write_unit(s, outv, osem):
        return pltpu.async_copy(outv, out_hbm.at[s, :, w], osem)

    def drain_write(s, outv, osem):
        pltpu.make_async_copy(outv, out_hbm.at[s, :, w], osem).wait()

    def process_batch(b0, rowbuf, first_par, skip_drains=0):
        for u in range(G):
            par = (first_par + u) % 2
            if u >= skip_drains:
                with jax.named_scope("dwr"):
                    drain_write(b0 + u, outvs[par], osems[par])
            with jax.named_scope("xpose"):
                transpose_unit(u, rowbuf, outvs[par])
            with jax.named_scope("wstart"):
                write_unit(b0 + u, outvs[par], osems[par])

    ga = gather_batch(0, stage_a, row_a, gsem_a)
    gather_batch(G, stage_b, row_b, gsem_b)
    ga.wait()
    process_batch(0, row_a, 0, skip_drains=2)

    def body(i, carry):
        b1 = 2 * i + 1
        gather_batch(G * (b1 + 1), stage_a, row_a, gsem_a)
        drain_gather(stage_b, row_b, gsem_b)
        process_batch(G * b1, row_b, 1)
        gather_batch(G * (b1 + 2), stage_b, row_b, gsem_b)
        drain_gather(stage_a, row_a, gsem_a)
        process_batch(G * (b1 + 1), row_a, 0)
        return carry

    lax.fori_loop(0, (NBATCH - 2) // 2, body, 0)

    drain_gather(stage_b, row_b, gsem_b)
    process_batch(G * (NBATCH - 1), row_b, 1)

    drain_write(S - 2, outvs[0], osems[0])
    drain_write(S - 1, outvs[1], osems[1])


def kernel(indices, table):
    idx = indices.reshape(-1).astype(jnp.int32)
    out5 = _gather_kernel(idx, table)
    return out5.transpose(2, 4, 0, 1, 3).reshape(BATCH, S, EMBED_DIM)


# speedup vs baseline: 1.0032x; 1.0032x over previous
import functools

import jax
import jax.numpy as jnp
from jax import lax
from jax.experimental import pallas as pl
from jax.experimental.pallas import tpu as pltpu
from jax.experimental.pallas import tpu_sc as plsc

EMBED_DIM = 64
S = 50
NUM_WORKERS = 32
BATCH = 4096
BB = BATCH // NUM_WORKERS
B_PER_W = BB * S
G = 5
NBATCH = S // G

_mesh = plsc.VectorSubcoreMesh(core_axis_name="c", subcore_axis_name="s")


@functools.partial(
    pl.kernel,
    mesh=_mesh,
    out_type=jax.ShapeDtypeStruct((S, 8, NUM_WORKERS, 8, 128), jnp.float32),
    scratch_types=(
        [pltpu.VMEM((B_PER_W,), jnp.int32)]
        + [pltpu.VMEM((G * BB,), jnp.int32) for _ in range(2)]
        + [pltpu.VMEM((G * BB, EMBED_DIM), jnp.float32) for _ in range(2)]
        + [pltpu.VMEM((EMBED_DIM, 128), jnp.float32) for _ in range(2)]
        + [pltpu.SemaphoreType.DMA for _ in range(4)]
    ),
    compiler_params=pltpu.CompilerParams(
        use_tc_tiling_on_sc=False, needs_layout_passes=False
    ),
)
def _gather_kernel(idx_hbm, table_hbm, out_hbm,
                   idx_v, stage_a, stage_b, row_a, row_b, outv_a, outv_b,
                   gsem_a, gsem_b, osem_a, osem_b):
    w = lax.axis_index("s") * 2 + lax.axis_index("c")

    pltpu.sync_copy(idx_hbm.at[pl.ds(w * B_PER_W, B_PER_W)], idx_v)

    iota = lax.iota(jnp.int32, 16)
    jm50 = [(jnp.full((16,), g * 16, jnp.int32) + iota) * S for g in range(8)]

    outvs = (outv_a, outv_b)
    osems = (osem_a, osem_b)

    def gather_batch(b0, stage, rowbuf, gsem):
        for u in range(G):
            s_splat = jnp.full((16,), b0 + u, jnp.int32)
            for g in range(8):
                v = plsc.load_gather(idx_v, [jm50[g] + s_splat])
                stage[pl.ds((u * 8 + g) * 16, 16)] = v
        return pltpu.async_copy(table_hbm.at[stage], rowbuf, gsem)

    def drain_gather(stage, rowbuf, gsem):
        pltpu.make_async_copy(table_hbm.at[stage], rowbuf, gsem).wait()

    def transpose_unit(u, rowbuf, outv):
        rows0 = tuple(
            jnp.full((16,), u * BB + g * 16, jnp.int32) + iota for g in range(8)
        )
        zero = jnp.zeros((16,), jnp.int32)

        @plsc.parallel_loop(0, EMBED_DIM, step=1, unroll=8,
                            carry=(zero, rows0))
        def body(d, c):
            cold, rows = c
            for g in range(8):
                v = plsc.load_gather(rowbuf, [rows[g], cold])
                outv[d, pl.ds(g * 16, 16)] = v
            return (cold + 1, rows)

    def write_unit(s, outv, osem):
        cps = []
        for db in range(8):
            cps.append(pltpu.async_copy(
                outv.at[pl.ds(db * 8, 8)], out_hbm.at[s, db, w], osem))
        return cps

    def drain_write(s, outv, osem):
        for db in range(8):
            pltpu.make_async_copy(
                outv.at[pl.ds(db * 8, 8)], out_hbm.at[s, db, w], osem).wait()

    def process_batch(b0, rowbuf, first_par, skip_drains=0):
        for u in range(G):
            par = (first_par + u) % 2
            if u >= skip_drains:
                drain_write(b0 + u, outvs[par], osems[par])
            transpose_unit(u, rowbuf, outvs[par])
            write_unit(b0 + u, outvs[par], osems[par])

    ga = gather_batch(0, stage_a, row_a, gsem_a)
    gather_batch(G, stage_b, row_b, gsem_b)
    ga.wait()
    process_batch(0, row_a, 0, skip_drains=2)

    def body(i, carry):
        b1 = 2 * i + 1
        gather_batch(G * (b1 + 1), stage_a, row_a, gsem_a)
        drain_gather(stage_b, row_b, gsem_b)
        process_batch(G * b1, row_b, 1)
        gather_batch(G * (b1 + 2), stage_b, row_b, gsem_b)
        drain_gather(stage_a, row_a, gsem_a)
        process_batch(G * (b1 + 1), row_a, 0)
        return carry

    lax.fori_loop(0, (NBATCH - 2) // 2, body, 0)

    drain_gather(stage_b, row_b, gsem_b)
    process_batch(G * (NBATCH - 1), row_b, 1)

    drain_write(S - 2, outvs[0], osems[0])
    drain_write(S - 1, outvs[1], osems[1])


def kernel(indices, table):
    idx = indices.reshape(-1).astype(jnp.int32)
    out5 = _gather_kernel(idx, table)
    return out5.transpose(2, 4, 0, 1, 3).reshape(BATCH, S, EMBED_DIM)


# speedup vs baseline: 1.0549x; 1.0516x over previous
import functools

import jax
import jax.numpy as jnp
from jax import lax
from jax.experimental import pallas as pl
from jax.experimental.pallas import tpu as pltpu
from jax.experimental.pallas import tpu_sc as plsc

EMBED_DIM = 64
S = 50
NUM_WORKERS = 32
BATCH = 4096
BB = BATCH // NUM_WORKERS
B_PER_W = BB * S
G = 5
NBATCH = S // G
TR_UNROLL = 4

_mesh = plsc.VectorSubcoreMesh(core_axis_name="c", subcore_axis_name="s")


@functools.partial(
    pl.kernel,
    mesh=_mesh,
    out_type=jax.ShapeDtypeStruct((S, 8, NUM_WORKERS, 8, 128), jnp.float32),
    scratch_types=(
        [pltpu.VMEM((B_PER_W,), jnp.int32)]
        + [pltpu.VMEM((G * BB,), jnp.int32) for _ in range(2)]
        + [pltpu.VMEM((G * BB, EMBED_DIM), jnp.float32) for _ in range(2)]
        + [pltpu.VMEM((EMBED_DIM, 128), jnp.float32) for _ in range(2)]
        + [pltpu.SemaphoreType.DMA for _ in range(4)]
    ),
    compiler_params=pltpu.CompilerParams(
        use_tc_tiling_on_sc=False, needs_layout_passes=False
    ),
)
def _gather_kernel(idx_hbm, table_hbm, out_hbm,
                   idx_v, stage_a, stage_b, row_a, row_b, outv_a, outv_b,
                   gsem_a, gsem_b, osem_a, osem_b):
    w = lax.axis_index("s") * 2 + lax.axis_index("c")

    pltpu.sync_copy(idx_hbm.at[pl.ds(w * B_PER_W, B_PER_W)], idx_v)

    iota = lax.iota(jnp.int32, 16)
    jm50 = [(jnp.full((16,), g * 16, jnp.int32) + iota) * S for g in range(8)]

    outvs = (outv_a, outv_b)
    osems = (osem_a, osem_b)

    def gather_batch(b0, stage, rowbuf, gsem):
        for u in range(G):
            s_splat = jnp.full((16,), b0 + u, jnp.int32)
            for g in range(8):
                v = plsc.load_gather(idx_v, [jm50[g] + s_splat])
                stage[pl.ds((u * 8 + g) * 16, 16)] = v
        return pltpu.async_copy(table_hbm.at[stage], rowbuf, gsem)

    def drain_gather(stage, rowbuf, gsem):
        pltpu.make_async_copy(table_hbm.at[stage], rowbuf, gsem).wait()

    def transpose_unit(u, rowbuf, outv):
        rows0 = tuple(
            jnp.full((16,), u * BB + g * 16, jnp.int32) + iota for g in range(8)
        )
        zero = jnp.zeros((16,), jnp.int32)

        @plsc.parallel_loop(0, EMBED_DIM, step=1, unroll=TR_UNROLL,
                            carry=(zero, rows0))
        def body(d, c):
            cold, rows = c
            for g in range(8):
                v = plsc.load_gather(rowbuf, [rows[g], cold])
                outv[d, pl.ds(g * 16, 16)] = v
            return (cold + 1, rows)

    def write_unit(s, outv, osem):
        cps = []
        for db in range(8):
            cps.append(pltpu.async_copy(
                outv.at[pl.ds(db * 8, 8)], out_hbm.at[s, db, w], osem))
        return cps

    def drain_write(s, outv, osem):
        for db in range(8):
            pltpu.make_async_copy(
                outv.at[pl.ds(db * 8, 8)], out_hbm.at[s, db, w], osem).wait()

    def process_batch(b0, rowbuf, first_par, skip_drains=0):
        for u in range(G):
            par = (first_par + u) % 2
            if u >= skip_drains:
                drain_write(b0 + u, outvs[par], osems[par])
            transpose_unit(u, rowbuf, outvs[par])
            write_unit(b0 + u, outvs[par], osems[par])

    ga = gather_batch(0, stage_a, row_a, gsem_a)
    gather_batch(G, stage_b, row_b, gsem_b)
    ga.wait()
    process_batch(0, row_a, 0, skip_drains=2)

    def body(i, carry):
        b1 = 2 * i + 1
        gather_batch(G * (b1 + 1), stage_a, row_a, gsem_a)
        drain_gather(stage_b, row_b, gsem_b)
        process_batch(G * b1, row_b, 1)
        gather_batch(G * (b1 + 2), stage_b, row_b, gsem_b)
        drain_gather(stage_a, row_a, gsem_a)
        process_batch(G * (b1 + 1), row_a, 0)
        return carry

    lax.fori_loop(0, (NBATCH - 2) // 2, body, 0)

    drain_gather(stage_b, row_b, gsem_b)
    process_batch(G * (NBATCH - 1), row_b, 1)

    drain_write(S - 2, outvs[0], osems[0])
    drain_write(S - 1, outvs[1], osems[1])


def kernel(indices, table):
    idx = indices.reshape(-1).astype(jnp.int32)
    out5 = _gather_kernel(idx, table)
    return out5.transpose(2, 4, 0, 1, 3).reshape(BATCH, S, EMBED_DIM)


# speedup vs baseline: 2.1230x; 2.0124x over previous
import functools

import jax
import jax.numpy as jnp
from jax import lax
from jax.experimental import pallas as pl
from jax.experimental.pallas import tpu as pltpu
from jax.experimental.pallas import tpu_sc as plsc

EMBED_DIM = 64
S = 50
NUM_WORKERS = 32
BATCH = 4096
BB = BATCH // NUM_WORKERS
B_PER_W = BB * S
G = 5
NBATCH = S // G
TR_UNROLL = 4

_mesh = plsc.VectorSubcoreMesh(core_axis_name="c", subcore_axis_name="s")


@functools.partial(
    pl.kernel,
    mesh=_mesh,
    out_type=jax.ShapeDtypeStruct((S, 8, NUM_WORKERS, 8, 128), jnp.float32),
    scratch_types=(
        [pltpu.VMEM((B_PER_W,), jnp.int32)]
        + [pltpu.VMEM((G * BB,), jnp.int32) for _ in range(2)]
        + [pltpu.VMEM((G * BB, EMBED_DIM), jnp.float32) for _ in range(2)]
        + [pltpu.VMEM((EMBED_DIM, 129), jnp.float32) for _ in range(2)]
        + [pltpu.SemaphoreType.DMA for _ in range(4)]
    ),
    compiler_params=pltpu.CompilerParams(
        use_tc_tiling_on_sc=False, needs_layout_passes=False
    ),
)
def _gather_kernel(idx_hbm, table_hbm, out_hbm,
                   idx_v, stage_a, stage_b, row_a, row_b, outv_a, outv_b,
                   gsem_a, gsem_b, osem_a, osem_b):
    w = lax.axis_index("s") * 2 + lax.axis_index("c")

    pltpu.sync_copy(idx_hbm.at[pl.ds(w * B_PER_W, B_PER_W)], idx_v)

    iota = lax.iota(jnp.int32, 16)
    jm50 = [(jnp.full((16,), g * 16, jnp.int32) + iota) * S for g in range(8)]

    outvs = (outv_a, outv_b)
    osems = (osem_a, osem_b)

    def gather_batch(b0, stage, rowbuf, gsem):
        for u in range(G):
            s_splat = jnp.full((16,), b0 + u, jnp.int32)
            for g in range(8):
                v = plsc.load_gather(idx_v, [jm50[g] + s_splat])
                stage[pl.ds((u * 8 + g) * 16, 16)] = v
        return pltpu.async_copy(table_hbm.at[stage], rowbuf, gsem)

    def drain_gather(stage, rowbuf, gsem):
        pltpu.make_async_copy(table_hbm.at[stage], rowbuf, gsem).wait()

    d16s = tuple(
        jnp.full((16,), dg * 16, jnp.int32) + iota for dg in range(4)
    )

    def transpose_unit(u, rowbuf, outv):
        @plsc.parallel_loop(0, BB, step=1, unroll=TR_UNROLL, carry=d16s)
        def body(b, d16c):
            bs = jnp.full((16,), b, jnp.int32)
            for dg in range(4):
                v = rowbuf[u * BB + b, pl.ds(dg * 16, 16)]
                plsc.store_scatter(outv, [d16c[dg], bs], v)
            return d16c

    def write_unit(s, outv, osem):
        cps = []
        for db in range(8):
            cps.append(pltpu.async_copy(
                outv.at[pl.ds(db * 8, 8), pl.ds(0, 128)],
                out_hbm.at[s, db, w], osem))
        return cps

    def drain_write(s, outv, osem):
        for db in range(8):
            pltpu.make_async_copy(
                outv.at[pl.ds(db * 8, 8), pl.ds(0, 128)],
                out_hbm.at[s, db, w], osem).wait()

    def process_batch(b0, rowbuf, first_par, skip_drains=0):
        for u in range(G):
            par = (first_par + u) % 2
            if u >= skip_drains:
                drain_write(b0 + u, outvs[par], osems[par])
            transpose_unit(u, rowbuf, outvs[par])
            write_unit(b0 + u, outvs[par], osems[par])

    ga = gather_batch(0, stage_a, row_a, gsem_a)
    gather_batch(G, stage_b, row_b, gsem_b)
    ga.wait()
    process_batch(0, row_a, 0, skip_drains=2)

    def body(i, carry):
        b1 = 2 * i + 1
        gather_batch(G * (b1 + 1), stage_a, row_a, gsem_a)
        drain_gather(stage_b, row_b, gsem_b)
        process_batch(G * b1, row_b, 1)
        gather_batch(G * (b1 + 2), stage_b, row_b, gsem_b)
        drain_gather(stage_a, row_a, gsem_a)
        process_batch(G * (b1 + 1), row_a, 0)
        return carry

    lax.fori_loop(0, (NBATCH - 2) // 2, body, 0)

    drain_gather(stage_b, row_b, gsem_b)
    process_batch(G * (NBATCH - 1), row_b, 1)

    drain_write(S - 2, outvs[0], osems[0])
    drain_write(S - 1, outvs[1], osems[1])


def kernel(indices, table):
    idx = indices.reshape(-1).astype(jnp.int32)
    out5 = _gather_kernel(idx, table)
    return out5.transpose(2, 4, 0, 1, 3).reshape(BATCH, S, EMBED_DIM)


# speedup vs baseline: 2.1278x; 1.0023x over previous
import functools

import jax
import jax.numpy as jnp
from jax import lax
from jax.experimental import pallas as pl
from jax.experimental.pallas import tpu as pltpu
from jax.experimental.pallas import tpu_sc as plsc

EMBED_DIM = 64
S = 50
NUM_WORKERS = 32
BATCH = 4096
BB = BATCH // NUM_WORKERS
B_PER_W = BB * S
G = 5
NBATCH = S // G
TR_UNROLL = 8

_mesh = plsc.VectorSubcoreMesh(core_axis_name="c", subcore_axis_name="s")


@functools.partial(
    pl.kernel,
    mesh=_mesh,
    out_type=jax.ShapeDtypeStruct((S, 8, NUM_WORKERS, 8, 128), jnp.float32),
    scratch_types=(
        [pltpu.VMEM((B_PER_W,), jnp.int32)]
        + [pltpu.VMEM((G * BB,), jnp.int32) for _ in range(2)]
        + [pltpu.VMEM((G * BB, EMBED_DIM), jnp.float32) for _ in range(2)]
        + [pltpu.VMEM((EMBED_DIM, 129), jnp.float32) for _ in range(2)]
        + [pltpu.SemaphoreType.DMA for _ in range(4)]
    ),
    compiler_params=pltpu.CompilerParams(
        use_tc_tiling_on_sc=False, needs_layout_passes=False
    ),
)
def _gather_kernel(idx_hbm, table_hbm, out_hbm,
                   idx_v, stage_a, stage_b, row_a, row_b, outv_a, outv_b,
                   gsem_a, gsem_b, osem_a, osem_b):
    w = lax.axis_index("s") * 2 + lax.axis_index("c")

    pltpu.sync_copy(idx_hbm.at[pl.ds(w * B_PER_W, B_PER_W)], idx_v)

    iota = lax.iota(jnp.int32, 16)
    jm50 = [(jnp.full((16,), g * 16, jnp.int32) + iota) * S for g in range(8)]

    outvs = (outv_a, outv_b)
    osems = (osem_a, osem_b)

    def gather_batch(b0, stage, rowbuf, gsem):
        for u in range(G):
            s_splat = jnp.full((16,), b0 + u, jnp.int32)
            for g in range(8):
                v = plsc.load_gather(idx_v, [jm50[g] + s_splat])
                stage[pl.ds((u * 8 + g) * 16, 16)] = v
        return pltpu.async_copy(table_hbm.at[stage], rowbuf, gsem)

    def drain_gather(stage, rowbuf, gsem):
        pltpu.make_async_copy(table_hbm.at[stage], rowbuf, gsem).wait()

    d16s = tuple(
        jnp.full((16,), dg * 16, jnp.int32) + iota for dg in range(4)
    )

    def transpose_unit(u, rowbuf, outv):
        @plsc.parallel_loop(0, BB, step=1, unroll=TR_UNROLL, carry=d16s)
        def body(b, d16c):
            bs = jnp.full((16,), b, jnp.int32)
            for dg in range(4):
                v = rowbuf[u * BB + b, pl.ds(dg * 16, 16)]
                plsc.store_scatter(outv, [d16c[dg], bs], v)
            return d16c

    def write_unit(s, outv, osem):
        cps = []
        for db in range(8):
            cps.append(pltpu.async_copy(
                outv.at[pl.ds(db * 8, 8), pl.ds(0, 128)],
                out_hbm.at[s, db, w], osem))
        return cps

    def drain_write(s, outv, osem):
        for db in range(8):
            pltpu.make_async_copy(
                outv.at[pl.ds(db * 8, 8), pl.ds(0, 128)],
                out_hbm.at[s, db, w], osem).wait()

    def process_batch(b0, rowbuf, first_par, skip_drains=0):
        for u in range(G):
            par = (first_par + u) % 2
            if u >= skip_drains:
                drain_write(b0 + u, outvs[par], osems[par])
            transpose_unit(u, rowbuf, outvs[par])
            write_unit(b0 + u, outvs[par], osems[par])

    ga = gather_batch(0, stage_a, row_a, gsem_a)
    gather_batch(G, stage_b, row_b, gsem_b)
    ga.wait()
    process_batch(0, row_a, 0, skip_drains=2)

    def body(i, carry):
        b1 = 2 * i + 1
        gather_batch(G * (b1 + 1), stage_a, row_a, gsem_a)
        drain_gather(stage_b, row_b, gsem_b)
        process_batch(G * b1, row_b, 1)
        gather_batch(G * (b1 + 2), stage_b, row_b, gsem_b)
        drain_gather(stage_a, row_a, gsem_a)
        process_batch(G * (b1 + 1), row_a, 0)
        return carry

    lax.fori_loop(0, (NBATCH - 2) // 2, body, 0)

    drain_gather(stage_b, row_b, gsem_b)
    process_batch(G * (NBATCH - 1), row_b, 1)

    drain_write(S - 2, outvs[0], osems[0])
    drain_write(S - 1, outvs[1], osems[1])


def kernel(indices, table):
    idx = indices.reshape(-1).astype(jnp.int32)
    out5 = _gather_kernel(idx, table)
    return out5.transpose(2, 4, 0, 1, 3).reshape(BATCH, S, EMBED_DIM)
